# baseline (speedup 1.0000x reference)
import functools

import jax
import jax.numpy as jnp
from jax import lax
from jax.experimental import pallas as pl
from jax.experimental.pallas import tpu as pltpu
from jax.experimental.pallas import tpu_sc as plsc

_V = 1000
_N = 51200
_NC = 2
_NS = 16
_NW = _NC * _NS
_CW = 32
_K = 1024
_NK = _N // _K
_KBUF = 2
_ROWS_W = _N // _NW


def _lse_body(t_ref, o_ref):
    x = t_ref[...]
    m = jnp.max(x, axis=1, keepdims=True)
    s = jnp.sum(jnp.exp(x - m), axis=1, keepdims=True)
    o_ref[...] = m + jnp.log(s)


def _finalize_body(p_ref, o_ref):
    o_ref[...] = (jnp.sum(p_ref[...]) * (1.0 / _N)).reshape(1, 1)


def _make_sc_colgather():
    mesh = plsc.VectorSubcoreMesh(core_axis_name="c", subcore_axis_name="s")

    @functools.partial(
        pl.kernel,
        mesh=mesh,
        compiler_params=pltpu.CompilerParams(needs_layout_passes=False),
        out_type=[jax.ShapeDtypeStruct((_V, _N), jnp.float32),
                  jax.ShapeDtypeStruct((_NW, 16), jnp.float32)],
        scratch_types=(
            [pltpu.VMEM((_CW * _V,), jnp.float32),
             pltpu.VMEM((_ROWS_W,), jnp.int32),
             pltpu.VMEM((_ROWS_W,), jnp.int32),
             pltpu.VMEM((_ROWS_W,), jnp.float32),
             pltpu.VMEM((_ROWS_W,), jnp.float32),
             pltpu.VMEM((16,), jnp.float32),
             pltpu.VMEM((_K,), jnp.int32),
             pltpu.VMEM((_K,), jnp.int32)]
            + [pltpu.VMEM((_CW, _K), jnp.float32) for _ in range(_KBUF)]
            + [pltpu.SemaphoreType.DMA for _ in range(3)]
            + [pltpu.SemaphoreType.DMA for _ in range(2 * _KBUF)]
        ),
    )
    def sc_colgather(idx_hbm, tgt_hbm, lse_hbm, tflat_hbm, ttabf_hbm,
                     out_hbm, part_hbm,
                     tt_v, idx_v, fidx_v, lsei_v, tel_v, acc_v,
                     idxc0, idxc1, *rest):
        bufs = rest[:_KBUF]
        esem1, esem2, tsem = rest[_KBUF:_KBUF + 3]
        isems = rest[_KBUF + 3:_KBUF + 3 + _KBUF]
        wsems = rest[_KBUF + 3 + _KBUF:_KBUF + 3 + 2 * _KBUF]
        idxcs = (idxc0, idxc1)

        wid = lax.axis_index("s") * _NC + lax.axis_index("c")

        base = wid * _ROWS_W
        pltpu.sync_copy(idx_hbm.at[pl.ds(base, _ROWS_W)], idx_v)
        pltpu.sync_copy(tgt_hbm.at[pl.ds(base, _ROWS_W)], fidx_v)

        def fidx_body(i, carry):
            p = i * 16
            fidx_v[pl.ds(p, 16)] = (fidx_v[pl.ds(p, 16)]
                                    + idx_v[pl.ds(p, 16)] * _V)
            return carry

        lax.fori_loop(0, _ROWS_W // 16, fidx_body, 0)
        e1 = pltpu.make_async_copy(lse_hbm.at[idx_v], lsei_v, esem1)
        e1.start()
        e2 = pltpu.make_async_copy(tflat_hbm.at[fidx_v], tel_v, esem2)
        e2.start()

        c0 = jnp.minimum(wid * _CW, _V - _CW)
        pltpu.async_copy(ttabf_hbm.at[pl.ds(c0 * _V, _CW * _V)], tt_v,
                         tsem).wait()

        def idx_desc(k, j):
            return pltpu.make_async_copy(
                idx_hbm.at[pl.ds(k * _K, _K)], idxcs[j], isems[j])

        def write_desc(k, j):
            return pltpu.make_async_copy(
                bufs[j],
                out_hbm.at[pl.ds(c0, _CW), pl.ds(k * _K, _K)], wsems[j])

        def do_chunk(k, j):
            idx_desc(k, j).wait()

            @pl.when(k >= _KBUF)
            def _():
                write_desc(k, j).wait()

            def kv_body(kv, carry):
                p = kv * 16
                iv = idxcs[j][pl.ds(p, 16)]
                for cc in range(_CW):
                    bufs[j][cc, pl.ds(p, 16)] = plsc.load_gather(
                        tt_v, [iv + cc * _V])
                return carry

            lax.fori_loop(0, _K // 16, kv_body, 0)
            write_desc(k, j).start()
            @pl.when(k + _KBUF < _NK)
            def _():
                idx_desc(k + _KBUF, j).start()

        idx_desc(0, 0).start()
        idx_desc(1, 1).start()

        def body(i, carry):
            k0 = i * _KBUF
            for j in range(_KBUF):
                do_chunk(k0 + j, j)
            return carry

        lax.fori_loop(0, _NK // _KBUF, body, 0)
        for t in range(_NK - _KBUF, _NK):
            write_desc(t, t % _KBUF).wait()

        e1.wait()
        e2.wait()
        acc_v[...] = jnp.zeros((16,), jnp.float32)

        def acc_body(i, carry):
            p = i * 16
            acc_v[...] = acc_v[...] + (lsei_v[pl.ds(p, 16)]
                                       - tel_v[pl.ds(p, 16)])
            return carry

        lax.fori_loop(0, _ROWS_W // 16, acc_body, 0)
        pltpu.sync_copy(acc_v, part_hbm.at[wid])

    return sc_colgather


_sc_colgather = _make_sc_colgather()


def kernel(idx, targets, table):
    idxf = idx.reshape(-1).astype(jnp.int32)
    tgtf = targets.reshape(-1).astype(jnp.int32)
    lse = pl.pallas_call(
        _lse_body,
        out_shape=jax.ShapeDtypeStruct((_V, 1), jnp.float32),
    )(table).reshape(_V)
    tflat = jnp.pad(table.reshape(-1), (0, 8))
    ttabf = table.T.reshape(-1)
    logits2d_t, part = _sc_colgather(idxf, tgtf, lse, tflat, ttabf)
    logits2d = logits2d_t.T
    loss = pl.pallas_call(
        _finalize_body,
        out_shape=jax.ShapeDtypeStruct((1, 1), jnp.float32),
    )(part)
    return (logits2d, loss.reshape(()))


# speedup vs baseline: 2.9956x; 2.9956x over previous
import functools

import jax
import jax.numpy as jnp
from jax import lax
from jax.experimental import pallas as pl
from jax.experimental.pallas import tpu as pltpu
from jax.experimental.pallas import tpu_sc as plsc

_V = 1000
_N = 51200
_NC = 2
_NS = 16
_NW = _NC * _NS
_CW = 32
_K = 1024
_NK = _N // _K
_KBUF = 2
_ROWS_W = _N // _NW


def _lse_body(t_ref, o_ref, tt_ref):
    x = t_ref[...]
    m = jnp.max(x, axis=1, keepdims=True)
    s = jnp.sum(jnp.exp(x - m), axis=1, keepdims=True)
    o_ref[...] = m + jnp.log(s)
    tt_ref[...] = x.T


def _finalize_body(p_ref, o_ref):
    o_ref[...] = (jnp.sum(p_ref[...]) * (1.0 / _N)).reshape(1, 1)


def _make_sc_colgather():
    mesh = plsc.VectorSubcoreMesh(core_axis_name="c", subcore_axis_name="s")

    @functools.partial(
        pl.kernel,
        mesh=mesh,
        compiler_params=pltpu.CompilerParams(needs_layout_passes=False),
        out_type=[jax.ShapeDtypeStruct((_V, _N), jnp.float32),
                  jax.ShapeDtypeStruct((_NW, 16), jnp.float32)],
        scratch_types=(
            [pltpu.VMEM((_CW * _V,), jnp.float32),
             pltpu.VMEM((_ROWS_W,), jnp.int32),
             pltpu.VMEM((_ROWS_W,), jnp.int32),
             pltpu.VMEM((_ROWS_W,), jnp.float32),
             pltpu.VMEM((_ROWS_W,), jnp.float32),
             pltpu.VMEM((16,), jnp.float32)]
            + [pltpu.VMEM((_K,), jnp.int32) for _ in range(_KBUF)]
            + [pltpu.VMEM((_CW, _K), jnp.float32) for _ in range(_KBUF)]
            + [pltpu.SemaphoreType.DMA for _ in range(3)]
            + [pltpu.SemaphoreType.DMA for _ in range(2 * _KBUF)]
        ),
    )
    def sc_colgather(idx_hbm, tgt_hbm, lse_hbm, ttabf_hbm,
                     out_hbm, part_hbm,
                     tt_v, idx_v, fidx_v, lsei_v, tel_v, acc_v,
                     *rest):
        idxcs = rest[:_KBUF]
        bufs = rest[_KBUF:2 * _KBUF]
        esem1, esem2, tsem = rest[2 * _KBUF:2 * _KBUF + 3]
        isems = rest[2 * _KBUF + 3:2 * _KBUF + 3 + _KBUF]
        wsems = rest[2 * _KBUF + 3 + _KBUF:2 * _KBUF + 3 + 2 * _KBUF]

        wid = lax.axis_index("s") * _NC + lax.axis_index("c")

        base = wid * _ROWS_W
        pltpu.sync_copy(idx_hbm.at[pl.ds(base, _ROWS_W)], idx_v)
        pltpu.sync_copy(tgt_hbm.at[pl.ds(base, _ROWS_W)], fidx_v)

        def fidx_body(i, carry):
            p = i * 16
            fidx_v[pl.ds(p, 16)] = (fidx_v[pl.ds(p, 16)] * _V
                                    + idx_v[pl.ds(p, 16)])
            return carry

        lax.fori_loop(0, _ROWS_W // 16, fidx_body, 0)
        e1 = pltpu.make_async_copy(lse_hbm.at[idx_v], lsei_v, esem1)
        e1.start()
        e2 = pltpu.make_async_copy(ttabf_hbm.at[fidx_v], tel_v, esem2)
        e2.start()

        c0 = jnp.minimum(wid * _CW, _V - _CW)
        pltpu.async_copy(ttabf_hbm.at[pl.ds(c0 * _V, _CW * _V)], tt_v,
                         tsem).wait()

        def idx_desc(k, j):
            return pltpu.make_async_copy(
                idx_hbm.at[pl.ds(k * _K, _K)], idxcs[j], isems[j])

        def write_desc(k, j):
            return pltpu.make_async_copy(
                bufs[j],
                out_hbm.at[pl.ds(c0, _CW), pl.ds(k * _K, _K)], wsems[j])

        def do_chunk(k, j):
            idx_desc(k, j).wait()

            @pl.when(k >= _KBUF)
            def _():
                write_desc(k, j).wait()

            @plsc.parallel_loop(0, _K // 16, unroll=2)
            def kv_body(kv):
                p = kv * 16
                iv = idxcs[j][pl.ds(p, 16)]
                for cc in range(_CW):
                    bufs[j][cc, pl.ds(p, 16)] = plsc.load_gather(
                        tt_v, [iv + cc * _V])

            write_desc(k, j).start()
            @pl.when(k + _KBUF < _NK)
            def _():
                idx_desc(k + _KBUF, j).start()

        for p in range(_KBUF):
            idx_desc(p, p).start()

        def body(i, carry):
            k0 = i * _KBUF
            for j in range(_KBUF):
                do_chunk(k0 + j, j)
            return carry

        lax.fori_loop(0, _NK // _KBUF, body, 0)
        for t in range(_NK - _KBUF, _NK):
            write_desc(t, t % _KBUF).wait()

        e1.wait()
        e2.wait()
        acc_v[...] = jnp.zeros((16,), jnp.float32)

        def acc_body(i, carry):
            p = i * 16
            acc_v[...] = acc_v[...] + (lsei_v[pl.ds(p, 16)]
                                       - tel_v[pl.ds(p, 16)])
            return carry

        lax.fori_loop(0, _ROWS_W // 16, acc_body, 0)
        pltpu.sync_copy(acc_v, part_hbm.at[wid])

    return sc_colgather


_sc_colgather = _make_sc_colgather()


def kernel(idx, targets, table):
    idxf = idx.reshape(-1).astype(jnp.int32)
    tgtf = targets.reshape(-1).astype(jnp.int32)
    lse2d, ttab = pl.pallas_call(
        _lse_body,
        out_shape=[jax.ShapeDtypeStruct((_V, 1), jnp.float32),
                   jax.ShapeDtypeStruct((_V, _V), jnp.float32)],
    )(table)
    lse = lse2d.reshape(_V)
    ttabf = ttab.reshape(-1)
    logits2d_t, part = _sc_colgather(idxf, tgtf, lse, ttabf)
    logits2d = logits2d_t.T
    loss = pl.pallas_call(
        _finalize_body,
        out_shape=jax.ShapeDtypeStruct((1, 1), jnp.float32),
    )(part)
    return (logits2d, loss.reshape(()))
